# baseline (speedup 1.0000x reference)
import functools

import jax
import jax.numpy as jnp
from jax import lax
from jax.experimental import pallas as pl
from jax.experimental.pallas import tpu as pltpu
from jax.experimental.pallas import tpu_sc as plsc

_HIDDEN = 1024
_INTER = 4096
_E = 8
_K = 2
_AUX_COEF = 0.01

_TM = 512
_TN = 512

_NW = 32
_CHUNK = 32


def _sc_gather_rows(table, idx):
    B = idx.shape[0]
    D = table.shape[1]
    b_per_w = B // _NW
    n_chunks = b_per_w // _CHUNK
    mesh = plsc.VectorSubcoreMesh(core_axis_name="c", subcore_axis_name="s")

    @functools.partial(
        pl.kernel,
        mesh=mesh,
        out_type=jax.ShapeDtypeStruct((B, D), jnp.float32),
        scratch_types=[
            pltpu.VMEM((b_per_w,), jnp.int32),
            pltpu.VMEM((2, _CHUNK, D), jnp.float32),
            pltpu.SemaphoreType.DMA,
            pltpu.SemaphoreType.DMA,
        ],
    )
    def gk(table_hbm, idx_hbm, out_hbm, idx_v, rows_v, sem_a, sem_b):
        wid = lax.axis_index("s") * 2 + lax.axis_index("c")
        base = wid * b_per_w
        pltpu.sync_copy(idx_hbm.at[pl.ds(base, b_per_w)], idx_v)
        sems = [sem_a, sem_b]
        copies = [None, None]
        copies[0] = pltpu.async_copy(
            table_hbm.at[idx_v.at[pl.ds(0, _CHUNK)]], rows_v.at[0], sems[0])
        for i in range(n_chunks):
            bb = i % 2
            if i + 1 < n_chunks:
                nb = (i + 1) % 2
                copies[nb] = pltpu.async_copy(
                    table_hbm.at[idx_v.at[pl.ds((i + 1) * _CHUNK, _CHUNK)]],
                    rows_v.at[nb], sems[nb])
            copies[bb].wait()
            pltpu.sync_copy(rows_v.at[bb],
                            out_hbm.at[pl.ds(base + i * _CHUNK, _CHUNK)])

    return gk(table, idx)


def _ffn_body(te_ref, meff_ref, na_ref, x_ref, wg_ref, wu_ref, wd_ref,
              wr_ref, o_ref, acc_ref):
    m = pl.program_id(0)
    n = pl.program_id(1)

    @pl.when(m < na_ref[0])
    def _():
        @pl.when(n == 0)
        def _():
            acc_ref[...] = jnp.zeros_like(acc_ref)

        x = x_ref[...]
        g = jnp.dot(x, wg_ref[0], preferred_element_type=jnp.float32,
                    precision=jax.lax.Precision.DEFAULT)
        u = jnp.dot(x, wu_ref[0], preferred_element_type=jnp.float32,
                    precision=jax.lax.Precision.DEFAULT)
        a = (g * jax.nn.sigmoid(g)) * u
        acc_ref[...] += jnp.dot(a, wd_ref[0], preferred_element_type=jnp.float32,
                                precision=jax.lax.Precision.DEFAULT)

        @pl.when(n == pl.num_programs(1) - 1)
        def _():
            o_ref[...] = acc_ref[...] * wr_ref[...][:, :1]


def _pair_add_body(x_ref, o_ref):
    x = x_ref[...]
    o_ref[...] = x[:, :_HIDDEN] + x[:, _HIDDEN:]


def kernel(hidden_states, gate_w, w_gate, w_up, w_down):
    b, s, h = hidden_states.shape
    T = b * s
    P = T * _K
    flat = hidden_states.reshape(T, h)

    logits = flat @ gate_w
    probs = jax.nn.softmax(logits, axis=-1)
    lane = jnp.arange(_E, dtype=jnp.int32)[None, :]
    i1 = jnp.argmax(probs, axis=-1).astype(jnp.int32)
    w1 = jnp.max(probs, axis=-1)
    masked = jnp.where(lane == i1[:, None], -jnp.inf, probs)
    i2 = jnp.argmax(masked, axis=-1).astype(jnp.int32)
    w2 = jnp.max(masked, axis=-1)
    w = jnp.stack([w1, w2], axis=-1)
    idx = jnp.stack([i1, i2], axis=-1)
    w = w / jnp.sum(w, axis=-1, keepdims=True)

    flat_e = idx.reshape(-1).astype(jnp.int32)
    onehot = (flat_e[:, None] == jnp.arange(_E, dtype=jnp.int32)[None, :]
              ).astype(jnp.int32)
    cums = jnp.cumsum(onehot, axis=0)
    rank = jnp.take_along_axis(cums, flat_e[:, None], axis=1)[:, 0] - 1
    counts = cums[-1]
    p_mean = probs.mean(axis=0)
    aux_loss = _E * jnp.sum((counts.astype(jnp.float32) / T) * p_mean) * _AUX_COEF

    padded_sz = ((counts + _TM - 1) // _TM) * _TM
    pcsum = jnp.cumsum(padded_sz)
    padded_off = pcsum - padded_sz
    pos = padded_off[flat_e] + rank

    B_pad = P + _E * _TM
    num_m = B_pad // _TM
    src = jnp.zeros((B_pad,), jnp.int32).at[pos].set(
        jnp.arange(P, dtype=jnp.int32) // _K)

    x_pad = _sc_gather_rows(flat, src)

    w_flat = w.reshape(-1)
    w_rep = jnp.zeros((B_pad, 128), jnp.float32).at[pos].set(
        jnp.broadcast_to(w_flat[:, None], (P, 128)))

    m_ids = jnp.arange(num_m, dtype=jnp.int32)
    tile_start = m_ids * _TM
    tile_e = jnp.minimum(
        jnp.searchsorted(pcsum, tile_start, side="right").astype(jnp.int32),
        _E - 1)
    num_active = pcsum[-1] // _TM
    last = num_active - 1
    m_eff = jnp.minimum(m_ids, last)
    tile_e = jnp.where(m_ids < num_active, tile_e, tile_e[last])
    na_arr = num_active.reshape(1)

    y_pad = pl.pallas_call(
        _ffn_body,
        grid_spec=pltpu.PrefetchScalarGridSpec(
            num_scalar_prefetch=3,
            grid=(num_m, _INTER // _TN),
            in_specs=[
                pl.BlockSpec((_TM, h), lambda m, n, te, me, na: (me[m], 0)),
                pl.BlockSpec((1, h, _TN), lambda m, n, te, me, na: (te[m], 0, n)),
                pl.BlockSpec((1, h, _TN), lambda m, n, te, me, na: (te[m], 0, n)),
                pl.BlockSpec((1, _TN, h), lambda m, n, te, me, na: (te[m], n, 0)),
                pl.BlockSpec((_TM, 128), lambda m, n, te, me, na: (me[m], 0)),
            ],
            out_specs=pl.BlockSpec((_TM, h), lambda m, n, te, me, na: (me[m], 0)),
            scratch_shapes=[pltpu.VMEM((_TM, h), jnp.float32)],
        ),
        out_shape=jax.ShapeDtypeStruct((B_pad, h), jnp.float32),
        compiler_params=pltpu.CompilerParams(
            dimension_semantics=("arbitrary", "arbitrary")),
    )(tile_e, m_eff, na_arr, x_pad, w_gate, w_up, w_down, w_rep)

    yp = _sc_gather_rows(y_pad, pos)
    out = pl.pallas_call(
        _pair_add_body,
        grid=(T // 256,),
        in_specs=[pl.BlockSpec((256, 2 * h), lambda i: (i, 0))],
        out_specs=pl.BlockSpec((256, h), lambda i: (i, 0)),
        out_shape=jax.ShapeDtypeStruct((T, h), jnp.float32),
    )(yp.reshape(T, 2 * h))
    return out.reshape(b, s, h), aux_loss


# speedup vs baseline: 1.3147x; 1.3147x over previous
import jax
import jax.numpy as jnp
from jax.experimental import pallas as pl
from jax.experimental.pallas import tpu as pltpu

_HIDDEN = 1024
_INTER = 4096
_E = 8
_K = 2
_AUX_COEF = 0.01

_TM = 512
_TN = 512


def _ffn_body(te_ref, meff_ref, na_ref, x_ref, wg_ref, wu_ref, wd_ref,
              o_ref, acc_ref):
    m = pl.program_id(0)
    n = pl.program_id(1)

    @pl.when(m < na_ref[0])
    def _():
        @pl.when(n == 0)
        def _():
            acc_ref[...] = jnp.zeros_like(acc_ref)

        x = x_ref[...]
        g = jnp.dot(x, wg_ref[0], preferred_element_type=jnp.float32,
                    precision=jax.lax.Precision.DEFAULT)
        u = jnp.dot(x, wu_ref[0], preferred_element_type=jnp.float32,
                    precision=jax.lax.Precision.DEFAULT)
        a = (g * jax.nn.sigmoid(g)) * u
        acc_ref[...] += jnp.dot(a, wd_ref[0], preferred_element_type=jnp.float32,
                                precision=jax.lax.Precision.DEFAULT)

        @pl.when(n == pl.num_programs(1) - 1)
        def _():
            o_ref[...] = acc_ref[...]


def kernel(hidden_states, gate_w, w_gate, w_up, w_down):
    b, s, h = hidden_states.shape
    T = b * s
    P = T * _K
    flat = hidden_states.reshape(T, h)

    logits = flat @ gate_w
    probs = jax.nn.softmax(logits, axis=-1)
    lane = jnp.arange(_E, dtype=jnp.int32)[None, :]
    i1 = jnp.argmax(probs, axis=-1).astype(jnp.int32)
    w1 = jnp.max(probs, axis=-1)
    masked = jnp.where(lane == i1[:, None], -jnp.inf, probs)
    i2 = jnp.argmax(masked, axis=-1).astype(jnp.int32)
    w2 = jnp.max(masked, axis=-1)
    w = jnp.stack([w1, w2], axis=-1)
    idx = jnp.stack([i1, i2], axis=-1)
    w = w / jnp.sum(w, axis=-1, keepdims=True)

    flat_e = idx.reshape(-1).astype(jnp.int32)
    onehot = (flat_e[:, None] == jnp.arange(_E, dtype=jnp.int32)[None, :]
              ).astype(jnp.int32)
    cums = jnp.cumsum(onehot, axis=0)
    rank = jnp.take_along_axis(cums, flat_e[:, None], axis=1)[:, 0] - 1
    counts = cums[-1]
    p_mean = probs.mean(axis=0)
    aux_loss = _E * jnp.sum((counts.astype(jnp.float32) / T) * p_mean) * _AUX_COEF

    padded_sz = ((counts + _TM - 1) // _TM) * _TM
    pcsum = jnp.cumsum(padded_sz)
    padded_off = pcsum - padded_sz
    pos = padded_off[flat_e] + rank

    B_pad = P + _E * _TM
    num_m = B_pad // _TM
    src = jnp.zeros((B_pad,), jnp.int32).at[pos].set(
        jnp.arange(P, dtype=jnp.int32) // _K)
    x_pad = flat[src]

    m_ids = jnp.arange(num_m, dtype=jnp.int32)
    tile_start = m_ids * _TM
    tile_e = jnp.minimum(
        jnp.searchsorted(pcsum, tile_start, side="right").astype(jnp.int32),
        _E - 1)
    num_active = pcsum[-1] // _TM
    last = num_active - 1
    m_eff = jnp.minimum(m_ids, last)
    tile_e = jnp.where(m_ids < num_active, tile_e, tile_e[last])
    na_arr = num_active.reshape(1)

    y_pad = pl.pallas_call(
        _ffn_body,
        grid_spec=pltpu.PrefetchScalarGridSpec(
            num_scalar_prefetch=3,
            grid=(num_m, _INTER // _TN),
            in_specs=[
                pl.BlockSpec((_TM, h), lambda m, n, te, me, na: (me[m], 0)),
                pl.BlockSpec((1, h, _TN), lambda m, n, te, me, na: (te[m], 0, n)),
                pl.BlockSpec((1, h, _TN), lambda m, n, te, me, na: (te[m], 0, n)),
                pl.BlockSpec((1, _TN, h), lambda m, n, te, me, na: (te[m], n, 0)),
            ],
            out_specs=pl.BlockSpec((_TM, h), lambda m, n, te, me, na: (me[m], 0)),
            scratch_shapes=[pltpu.VMEM((_TM, h), jnp.float32)],
        ),
        out_shape=jax.ShapeDtypeStruct((B_pad, h), jnp.float32),
        compiler_params=pltpu.CompilerParams(
            dimension_semantics=("arbitrary", "arbitrary")),
    )(tile_e, m_eff, na_arr, x_pad, w_gate, w_up, w_down)

    yg = y_pad[pos].reshape(T, _K, h)
    out = jnp.sum(w[:, :, None] * yg, axis=1)
    return out.reshape(b, s, h), aux_loss


# speedup vs baseline: 1.3953x; 1.0613x over previous
import jax
import jax.numpy as jnp
from jax.experimental import pallas as pl
from jax.experimental.pallas import tpu as pltpu

_HIDDEN = 1024
_INTER = 4096
_E = 8
_K = 2
_AUX_COEF = 0.01

_TM = 512
_TN = 512


def _ffn_body(te_ref, meff_ref, na_ref, x_ref, wg_ref, wu_ref, wd_ref,
              o_ref, acc_ref):
    m = pl.program_id(0)
    n = pl.program_id(1)

    @pl.when(m < na_ref[0])
    def _():
        @pl.when(n == 0)
        def _():
            acc_ref[...] = jnp.zeros_like(acc_ref)

        x = x_ref[...]
        g = jnp.dot(x, wg_ref[0], preferred_element_type=jnp.float32,
                    precision=jax.lax.Precision.DEFAULT)
        u = jnp.dot(x, wu_ref[0], preferred_element_type=jnp.float32,
                    precision=jax.lax.Precision.DEFAULT)
        a = (g * jax.nn.sigmoid(g)) * u
        acc_ref[...] += jnp.dot(a, wd_ref[0], preferred_element_type=jnp.float32,
                                precision=jax.lax.Precision.DEFAULT)

        @pl.when(n == pl.num_programs(1) - 1)
        def _():
            o_ref[...] = acc_ref[...]


def kernel(hidden_states, gate_w, w_gate, w_up, w_down):
    b, s, h = hidden_states.shape
    T = b * s
    P = T * _K
    flat = hidden_states.reshape(T, h)

    logits = flat @ gate_w
    probs = jax.nn.softmax(logits, axis=-1)
    lane = jnp.arange(_E, dtype=jnp.int32)[None, :]
    i1 = jnp.argmax(probs, axis=-1).astype(jnp.int32)
    w1 = jnp.max(probs, axis=-1)
    masked = jnp.where(lane == i1[:, None], -jnp.inf, probs)
    i2 = jnp.argmax(masked, axis=-1).astype(jnp.int32)
    w2 = jnp.max(masked, axis=-1)
    w = jnp.stack([w1, w2], axis=-1)
    idx = jnp.stack([i1, i2], axis=-1)
    w = w / jnp.sum(w, axis=-1, keepdims=True)

    flat_e = idx.reshape(-1).astype(jnp.int32)
    onehot = (flat_e[:, None] == jnp.arange(_E, dtype=jnp.int32)[None, :]
              ).astype(jnp.int32)
    cums = jnp.cumsum(onehot, axis=0)
    rank = jnp.take_along_axis(cums, flat_e[:, None], axis=1)[:, 0] - 1
    counts = cums[-1]
    p_mean = probs.mean(axis=0)
    aux_loss = _E * jnp.sum((counts.astype(jnp.float32) / T) * p_mean) * _AUX_COEF

    padded_sz = ((counts + _TM - 1) // _TM) * _TM
    pcsum = jnp.cumsum(padded_sz)
    padded_off = pcsum - padded_sz
    pos = padded_off[flat_e] + rank

    B_pad = P + _E * _TM
    num_m = B_pad // _TM
    src = jnp.zeros((B_pad,), jnp.int32).at[pos].set(
        jnp.arange(P, dtype=jnp.int32) // _K)
    x_pad = flat[src]

    m_ids = jnp.arange(num_m, dtype=jnp.int32)
    tile_start = m_ids * _TM
    tile_e = jnp.minimum(
        jnp.searchsorted(pcsum, tile_start, side="right").astype(jnp.int32),
        _E - 1)
    num_active = pcsum[-1] // _TM
    last = num_active - 1
    m_eff = jnp.minimum(m_ids, last)
    tile_e = jnp.where(m_ids < num_active, tile_e, tile_e[last])
    na_arr = num_active.reshape(1)

    y_pad = pl.pallas_call(
        _ffn_body,
        grid_spec=pltpu.PrefetchScalarGridSpec(
            num_scalar_prefetch=3,
            grid=(num_m, _INTER // _TN),
            in_specs=[
                pl.BlockSpec((_TM, h), lambda m, n, te, me, na: (me[m], 0)),
                pl.BlockSpec((1, h, _TN), lambda m, n, te, me, na: (te[m], 0, n)),
                pl.BlockSpec((1, h, _TN), lambda m, n, te, me, na: (te[m], 0, n)),
                pl.BlockSpec((1, _TN, h), lambda m, n, te, me, na: (te[m], n, 0)),
            ],
            out_specs=pl.BlockSpec((_TM, h), lambda m, n, te, me, na: (me[m], 0)),
            scratch_shapes=[pltpu.VMEM((_TM, h), jnp.float32)],
        ),
        out_shape=jax.ShapeDtypeStruct((B_pad, h), jnp.float32),
        compiler_params=pltpu.CompilerParams(
            dimension_semantics=("arbitrary", "arbitrary")),
    )(tile_e, m_eff, na_arr, x_pad, w_gate, w_up, w_down)

    pos2 = pos.reshape(T, _K)
    out = (w[:, 0:1] * y_pad[pos2[:, 0]] + w[:, 1:2] * y_pad[pos2[:, 1]])
    return out.reshape(b, s, h), aux_loss


# speedup vs baseline: 1.4973x; 1.0730x over previous
import jax
import jax.numpy as jnp
from jax.experimental import pallas as pl
from jax.experimental.pallas import tpu as pltpu

_HIDDEN = 1024
_INTER = 4096
_E = 8
_K = 2
_AUX_COEF = 0.01

_TM = 512
_TN = 1024


def _ffn_body(te_ref, meff_ref, na_ref, x_ref, wg_ref, wu_ref, wd_ref,
              o_ref, acc_ref):
    m = pl.program_id(0)
    n = pl.program_id(1)

    @pl.when(m < na_ref[0])
    def _():
        @pl.when(n == 0)
        def _():
            acc_ref[...] = jnp.zeros_like(acc_ref)

        x = x_ref[...]
        g = jnp.dot(x, wg_ref[0], preferred_element_type=jnp.float32,
                    precision=jax.lax.Precision.DEFAULT)
        u = jnp.dot(x, wu_ref[0], preferred_element_type=jnp.float32,
                    precision=jax.lax.Precision.DEFAULT)
        a = (g * jax.nn.sigmoid(g)) * u
        acc_ref[...] += jnp.dot(a, wd_ref[0], preferred_element_type=jnp.float32,
                                precision=jax.lax.Precision.DEFAULT)

        @pl.when(n == pl.num_programs(1) - 1)
        def _():
            o_ref[...] = acc_ref[...]


def kernel(hidden_states, gate_w, w_gate, w_up, w_down):
    b, s, h = hidden_states.shape
    T = b * s
    P = T * _K
    flat = hidden_states.reshape(T, h)

    logits = flat @ gate_w
    probs = jax.nn.softmax(logits, axis=-1)
    lane = jnp.arange(_E, dtype=jnp.int32)[None, :]
    i1 = jnp.argmax(probs, axis=-1).astype(jnp.int32)
    w1 = jnp.max(probs, axis=-1)
    masked = jnp.where(lane == i1[:, None], -jnp.inf, probs)
    i2 = jnp.argmax(masked, axis=-1).astype(jnp.int32)
    w2 = jnp.max(masked, axis=-1)
    w = jnp.stack([w1, w2], axis=-1)
    idx = jnp.stack([i1, i2], axis=-1)
    w = w / jnp.sum(w, axis=-1, keepdims=True)

    flat_e = idx.reshape(-1).astype(jnp.int32)
    onehot = (flat_e[:, None] == jnp.arange(_E, dtype=jnp.int32)[None, :]
              ).astype(jnp.int32)
    cums = jnp.cumsum(onehot, axis=0)
    rank = jnp.take_along_axis(cums, flat_e[:, None], axis=1)[:, 0] - 1
    counts = cums[-1]
    p_mean = probs.mean(axis=0)
    aux_loss = _E * jnp.sum((counts.astype(jnp.float32) / T) * p_mean) * _AUX_COEF

    padded_sz = ((counts + _TM - 1) // _TM) * _TM
    pcsum = jnp.cumsum(padded_sz)
    padded_off = pcsum - padded_sz
    pos = padded_off[flat_e] + rank

    B_pad = P + _E * _TM
    num_m = B_pad // _TM
    src = jnp.zeros((B_pad,), jnp.int32).at[pos].set(
        jnp.arange(P, dtype=jnp.int32) // _K)
    x_pad = flat[src]

    m_ids = jnp.arange(num_m, dtype=jnp.int32)
    tile_start = m_ids * _TM
    tile_e = jnp.minimum(
        jnp.searchsorted(pcsum, tile_start, side="right").astype(jnp.int32),
        _E - 1)
    num_active = pcsum[-1] // _TM
    last = num_active - 1
    m_eff = jnp.minimum(m_ids, last)
    tile_e = jnp.where(m_ids < num_active, tile_e, tile_e[last])
    na_arr = num_active.reshape(1)

    y_pad = pl.pallas_call(
        _ffn_body,
        grid_spec=pltpu.PrefetchScalarGridSpec(
            num_scalar_prefetch=3,
            grid=(num_m, _INTER // _TN),
            in_specs=[
                pl.BlockSpec((_TM, h), lambda m, n, te, me, na: (me[m], 0)),
                pl.BlockSpec((1, h, _TN), lambda m, n, te, me, na: (te[m], 0, n)),
                pl.BlockSpec((1, h, _TN), lambda m, n, te, me, na: (te[m], 0, n)),
                pl.BlockSpec((1, _TN, h), lambda m, n, te, me, na: (te[m], n, 0)),
            ],
            out_specs=pl.BlockSpec((_TM, h), lambda m, n, te, me, na: (me[m], 0)),
            scratch_shapes=[pltpu.VMEM((_TM, h), jnp.float32)],
        ),
        out_shape=jax.ShapeDtypeStruct((B_pad, h), jnp.float32),
        compiler_params=pltpu.CompilerParams(
            dimension_semantics=("arbitrary", "arbitrary")),
    )(tile_e, m_eff, na_arr, x_pad, w_gate, w_up, w_down)

    pos2 = pos.reshape(T, _K)
    out = (w[:, 0:1] * y_pad[pos2[:, 0]] + w[:, 1:2] * y_pad[pos2[:, 1]])
    return out.reshape(b, s, h), aux_loss
